# baseline (speedup 1.0000x reference)
import functools

import jax
import jax.numpy as jnp
from jax import lax
from jax.experimental import pallas as pl
from jax.experimental.pallas import tpu as pltpu
from jax.experimental.pallas import tpu_sc as plsc

NUM_TOKENS = 16384
SEQ = 50
DIM = 32
NUM_WORKERS = 32
ROWS_PER_W = NUM_TOKENS // NUM_WORKERS
R = 32
N_CHUNKS = ROWS_PER_W // R
NBUF = 2

_mesh = plsc.VectorSubcoreMesh(core_axis_name="c", subcore_axis_name="s")

NUM_EMBED = 1000000
CB = 800
N_BLOCKS = NUM_EMBED // CB


@functools.partial(
    pl.kernel,
    mesh=_mesh,
    out_type=jax.ShapeDtypeStruct((NUM_EMBED, DIM), jnp.float32),
    scratch_types=[
        pltpu.VMEM((DIM, CB), jnp.float32),
        pltpu.VMEM((DIM, CB), jnp.float32),
        pltpu.VMEM((CB, DIM), jnp.float32),
        pltpu.VMEM((CB, DIM), jnp.float32),
        pltpu.SemaphoreType.DMA,
        pltpu.SemaphoreType.DMA,
        pltpu.SemaphoreType.DMA,
        pltpu.SemaphoreType.DMA,
    ],
    compiler_params=pltpu.CompilerParams(
        use_tc_tiling_on_sc=False, needs_layout_passes=False),
)
def _sc_transpose(wt_hbm, out_hbm, buf0, buf1, tbuf0, tbuf1, i0, i1, o0, o1):
    bufs = (buf0, buf1)
    tbufs = (tbuf0, tbuf1)
    isem = (i0, i1)
    osem = (o0, o1)
    wid = lax.axis_index("s") * 2 + lax.axis_index("c")
    lane = lax.broadcasted_iota(jnp.int32, (16,), 0)

    def blk_col(i):
        return (wid + i * NUM_WORKERS) * CB

    n_mine = N_BLOCKS // NUM_WORKERS
    rem = N_BLOCKS % NUM_WORKERS

    for b in range(NBUF):
        pltpu.async_copy(
            wt_hbm.at[:, pl.ds(blk_col(b), CB)], bufs[b], isem[b])

    def do_block(i, b):
        c0 = blk_col(i)
        pltpu.make_async_copy(
            wt_hbm.at[:, pl.ds(c0, CB)], bufs[b], isem[b]).wait()

        @pl.when(i >= NBUF)
        def _():
            pltpu.make_async_copy(
                tbufs[b], out_hbm.at[pl.ds(blk_col(i - NBUF), CB), :],
                osem[b]).wait()

        def col_body(cb, _):
            rows = cb * 16 + lane
            for f in range(DIM):
                v = bufs[b][f, pl.ds(cb * 16, 16)]
                plsc.store_scatter(
                    tbufs[b], [rows, jnp.full((16,), f, jnp.int32)], v)
            return ()

        lax.fori_loop(0, CB // 16, col_body, ())
        pltpu.async_copy(tbufs[b], out_hbm.at[pl.ds(c0, CB), :], osem[b])

        @pl.when(i + NBUF < n_mine + jnp.where(wid < rem, 1, 0))
        def _():
            pltpu.async_copy(
                wt_hbm.at[:, pl.ds(blk_col(i + NBUF), CB)],
                bufs[b], isem[b])

    def pair_body(p, _):
        for b in range(NBUF):
            do_block(p * NBUF + b, b)
        return ()

    n_total = n_mine + jnp.where(wid < rem, 1, 0)
    lax.fori_loop(0, n_total // NBUF, pair_body, ())

    @pl.when(n_total % NBUF == 1)
    def _():
        do_block(n_total - 1, 0)

    def drain(b, last):
        @pl.when(last - NBUF + b >= 0)
        def _():
            pltpu.make_async_copy(
                tbufs[b], out_hbm.at[pl.ds(blk_col(last - NBUF + b), CB), :],
                osem[b]).wait()

    for b in range(NBUF):
        drain(b, n_total)


@functools.partial(
    pl.kernel,
    mesh=_mesh,
    out_type=jax.ShapeDtypeStruct((NUM_TOKENS, SEQ, DIM), jnp.float32),
    scratch_types=[
        pltpu.VMEM((NBUF, R, SEQ), jnp.int32),
        pltpu.VMEM((NBUF, R, SEQ, DIM), jnp.float32),
        pltpu.SemaphoreType.DMA,
        pltpu.SemaphoreType.DMA,
        pltpu.SemaphoreType.DMA,
        pltpu.SemaphoreType.DMA,
        pltpu.SemaphoreType.DMA,
        pltpu.SemaphoreType.DMA,
    ],
    compiler_params=pltpu.CompilerParams(use_tc_tiling_on_sc=False),
)
def _sc_gather(idx_hbm, table_hbm, out_hbm, idx_v, rows_v,
               isem0, isem1, gsem0, gsem1, osem0, osem1):
    isem = (isem0, isem1)
    gsem = (gsem0, gsem1)
    osem = (osem0, osem1)
    wid = lax.axis_index("s") * 2 + lax.axis_index("c")
    row0 = wid * ROWS_PER_W

    def idx_src(g):
        return idx_hbm.at[pl.ds(row0 + g * R, R), :]

    def out_dst(g):
        return out_hbm.at[pl.ds(row0 + g * R, R), :, :]

    for b in range(NBUF):
        pltpu.async_copy(idx_src(b), idx_v.at[b], isem[b])

    def pair_body(p, _):
        for b in range(NBUF):
            g = p * NBUF + b
            pltpu.make_async_copy(
                idx_src(g), idx_v.at[b], isem[b]).wait()
            @pl.when(g >= NBUF)
            def _():
                pltpu.make_async_copy(
                    rows_v.at[b], out_dst(g - NBUF), osem[b]).wait()
            for j in range(R):
                pltpu.async_copy(
                    table_hbm.at[idx_v.at[b, j]],
                    rows_v.at[b, j], gsem[b])
            for j in range(R):
                pltpu.make_async_copy(
                    table_hbm.at[idx_v.at[b, j]],
                    rows_v.at[b, j], gsem[b]).wait()
            pltpu.async_copy(rows_v.at[b], out_dst(g), osem[b])
            @pl.when(g + NBUF < N_CHUNKS)
            def _():
                pltpu.async_copy(
                    idx_src(g + NBUF), idx_v.at[b], isem[b])
        return ()

    lax.fori_loop(0, N_CHUNKS // NBUF, pair_body, ())

    for b in range(NBUF):
        pltpu.make_async_copy(
            rows_v.at[b], out_dst(N_CHUNKS - NBUF + b), osem[b]).wait()


def kernel(token_ids, weight):
    w_lin = _sc_transpose(weight.T)
    return _sc_gather(token_ids, w_lin)


# speedup vs baseline: 2.9686x; 2.9686x over previous
import functools

import jax
import jax.numpy as jnp
from jax import lax
from jax.experimental import pallas as pl
from jax.experimental.pallas import tpu as pltpu
from jax.experimental.pallas import tpu_sc as plsc

NUM_TOKENS = 16384
SEQ = 50
DIM = 32
NUM_WORKERS = 32
BLK = NUM_TOKENS // NUM_WORKERS
NBUF = 2

_mesh = plsc.VectorSubcoreMesh(core_axis_name="c", subcore_axis_name="s")


@functools.partial(
    pl.kernel,
    mesh=_mesh,
    out_type=jax.ShapeDtypeStruct((SEQ, DIM, NUM_TOKENS), jnp.float32),
    scratch_types=[
        pltpu.VMEM((BLK, SEQ), jnp.int32),
        pltpu.VMEM((SEQ, BLK), jnp.int32),
        pltpu.VMEM((NBUF, BLK, DIM), jnp.float32),
        pltpu.VMEM((NBUF, DIM, BLK), jnp.float32),
        pltpu.SemaphoreType.DMA,
        pltpu.SemaphoreType.DMA,
        pltpu.SemaphoreType.DMA,
        pltpu.SemaphoreType.DMA,
        pltpu.SemaphoreType.DMA,
    ],
    compiler_params=pltpu.CompilerParams(
        use_tc_tiling_on_sc=False, needs_layout_passes=False),
)
def _sc_gather_t(idx_hbm, table_hbm, out_hbm, tok_in, tok_t, rows_v, tbuf_v,
                 tsem, gsem0, gsem1, osem0, osem1):
    gsem = (gsem0, gsem1)
    osem = (osem0, osem1)
    wid = lax.axis_index("s") * 2 + lax.axis_index("c")
    b0 = wid * BLK
    lane = lax.broadcasted_iota(jnp.int32, (16,), 0)

    pltpu.async_copy(idx_hbm.at[pl.ds(b0, BLK), :], tok_in, tsem)
    pltpu.make_async_copy(idx_hbm.at[pl.ds(b0, BLK), :], tok_in, tsem).wait()

    def tok_body(k, _):
        rows = k * 16 + lane
        for s in range(SEQ):
            v = plsc.load_gather(tok_in, [rows, jnp.full((16,), s, jnp.int32)])
            tok_t[s, pl.ds(k * 16, 16)] = v
        return ()

    lax.fori_loop(0, BLK // 16, tok_body, ())

    def gather_src(s):
        return table_hbm.at[tok_t.at[s]]

    for b in range(NBUF):
        pltpu.async_copy(gather_src(b), rows_v.at[b], gsem[b])

    def pair_body(p, _):
        for b in range(NBUF):
            s = p * NBUF + b
            pltpu.make_async_copy(gather_src(s), rows_v.at[b], gsem[b]).wait()

            @pl.when(s >= NBUF)
            def _():
                pltpu.make_async_copy(
                    tbuf_v.at[b],
                    out_hbm.at[s - NBUF, :, pl.ds(b0, BLK)], osem[b]).wait()

            def tr_body(k, _):
                rows = k * 16 + lane
                for d in range(DIM):
                    v = plsc.load_gather(
                        rows_v.at[b], [rows, jnp.full((16,), d, jnp.int32)])
                    tbuf_v[b, d, pl.ds(k * 16, 16)] = v
                return ()

            lax.fori_loop(0, BLK // 16, tr_body, ())
            pltpu.async_copy(
                tbuf_v.at[b], out_hbm.at[s, :, pl.ds(b0, BLK)], osem[b])

            @pl.when(s + NBUF < SEQ)
            def _():
                pltpu.async_copy(gather_src(s + NBUF), rows_v.at[b], gsem[b])
        return ()

    lax.fori_loop(0, SEQ // NBUF, pair_body, ())

    for b in range(NBUF):
        pltpu.make_async_copy(
            tbuf_v.at[b], out_hbm.at[SEQ - NBUF + b, :, pl.ds(b0, BLK)],
            osem[b]).wait()


def kernel(token_ids, weight):
    out_t = _sc_gather_t(token_ids, weight)
    return jnp.transpose(out_t, (2, 0, 1))


# speedup vs baseline: 3.8119x; 1.2841x over previous
import functools

import jax
import jax.numpy as jnp
from jax import lax
from jax.experimental import pallas as pl
from jax.experimental.pallas import tpu as pltpu
from jax.experimental.pallas import tpu_sc as plsc

NUM_TOKENS = 16384
SEQ = 50
DIM = 32
NUM_WORKERS = 32
BLK = NUM_TOKENS // NUM_WORKERS
NRING = 4

_mesh = plsc.VectorSubcoreMesh(core_axis_name="c", subcore_axis_name="s")


@functools.partial(
    pl.kernel,
    mesh=_mesh,
    out_type=jax.ShapeDtypeStruct((SEQ, NUM_TOKENS, DIM), jnp.float32),
    scratch_types=[
        pltpu.VMEM((BLK, SEQ), jnp.int32),
        pltpu.VMEM((SEQ, BLK), jnp.int32),
        pltpu.VMEM((NRING, BLK, DIM), jnp.float32),
        pltpu.SemaphoreType.DMA,
        pltpu.SemaphoreType.DMA,
        pltpu.SemaphoreType.DMA,
        pltpu.SemaphoreType.DMA,
        pltpu.SemaphoreType.DMA,
        pltpu.SemaphoreType.DMA,
        pltpu.SemaphoreType.DMA,
        pltpu.SemaphoreType.DMA,
        pltpu.SemaphoreType.DMA,
    ],
    compiler_params=pltpu.CompilerParams(
        use_tc_tiling_on_sc=False, needs_layout_passes=False),
)
def _sc_gather_t(idx_hbm, table_hbm, out_hbm, tok_in, tok_t, rows_v,
                 tsem, g0, g1, g2, g3, o0, o1, o2, o3):
    gsem = (g0, g1, g2, g3)
    osem = (o0, o1, o2, o3)
    wid = lax.axis_index("s") * 2 + lax.axis_index("c")
    b0 = wid * BLK
    lane = lax.broadcasted_iota(jnp.int32, (16,), 0)

    pltpu.async_copy(idx_hbm.at[pl.ds(b0, BLK), :], tok_in, tsem)
    pltpu.make_async_copy(idx_hbm.at[pl.ds(b0, BLK), :], tok_in, tsem).wait()

    def tok_body(k, _):
        rows = k * 16 + lane
        for s in range(SEQ):
            v = plsc.load_gather(tok_in, [rows, jnp.full((16,), s, jnp.int32)])
            tok_t[s, pl.ds(k * 16, 16)] = v
        return ()

    lax.fori_loop(0, BLK // 16, tok_body, ())

    def gather_src(s):
        return table_hbm.at[tok_t.at[s]]

    def out_dst(s):
        return out_hbm.at[s, pl.ds(b0, BLK), :]

    for q in range(2):
        pltpu.async_copy(gather_src(q), rows_v.at[q], gsem[q])

    def step(s, q):
        pltpu.make_async_copy(gather_src(s), rows_v.at[q], gsem[q]).wait()
        pltpu.async_copy(rows_v.at[q], out_dst(s), osem[q])
        q2 = (q + 2) % NRING

        @pl.when(s + 2 < SEQ)
        def _():
            @pl.when(s >= 2)
            def _():
                pltpu.make_async_copy(
                    rows_v.at[q2], out_dst(s - 2), osem[q2]).wait()
            pltpu.async_copy(gather_src(s + 2), rows_v.at[q2], gsem[q2])

    def quad_body(p, _):
        for q in range(NRING):
            step(p * NRING + q, q)
        return ()

    lax.fori_loop(0, SEQ // NRING, quad_body, ())
    for q in range(SEQ % NRING):
        step(SEQ - SEQ % NRING + q, q)

    for i in range(NRING):
        s = SEQ - NRING + i
        pltpu.make_async_copy(
            rows_v.at[s % NRING], out_dst(s), osem[s % NRING]).wait()


def kernel(token_ids, weight):
    out_t = _sc_gather_t(token_ids, weight)
    return jnp.transpose(out_t, (1, 0, 2))


# speedup vs baseline: 3.8653x; 1.0140x over previous
import functools

import jax
import jax.numpy as jnp
from jax import lax
from jax.experimental import pallas as pl
from jax.experimental.pallas import tpu as pltpu
from jax.experimental.pallas import tpu_sc as plsc

NUM_TOKENS = 16384
SEQ = 50
DIM = 32
NUM_WORKERS = 32
BLK = NUM_TOKENS // NUM_WORKERS
NRING = 4

_mesh = plsc.VectorSubcoreMesh(core_axis_name="c", subcore_axis_name="s")


@functools.partial(
    pl.kernel,
    mesh=_mesh,
    out_type=jax.ShapeDtypeStruct((SEQ, NUM_TOKENS, DIM), jnp.float32),
    scratch_types=[
        pltpu.VMEM((SEQ, BLK), jnp.int32),
        pltpu.VMEM((NRING, BLK, DIM), jnp.float32),
        pltpu.SemaphoreType.DMA,
        pltpu.SemaphoreType.DMA,
        pltpu.SemaphoreType.DMA,
        pltpu.SemaphoreType.DMA,
        pltpu.SemaphoreType.DMA,
        pltpu.SemaphoreType.DMA,
        pltpu.SemaphoreType.DMA,
        pltpu.SemaphoreType.DMA,
        pltpu.SemaphoreType.DMA,
    ],
    compiler_params=pltpu.CompilerParams(
        use_tc_tiling_on_sc=False, needs_layout_passes=False),
)
def _sc_gather_t(idx_hbm, table_hbm, out_hbm, tok_t, rows_v,
                 tsem, g0, g1, g2, g3, o0, o1, o2, o3):
    gsem = (g0, g1, g2, g3)
    osem = (o0, o1, o2, o3)
    wid = lax.axis_index("s") * 2 + lax.axis_index("c")
    b0 = wid * BLK

    pltpu.async_copy(idx_hbm.at[:, pl.ds(b0, BLK)], tok_t, tsem)
    pltpu.make_async_copy(idx_hbm.at[:, pl.ds(b0, BLK)], tok_t, tsem).wait()

    def gather_src(s):
        return table_hbm.at[tok_t.at[s]]

    def out_dst(s):
        return out_hbm.at[s, pl.ds(b0, BLK), :]

    for q in range(2):
        pltpu.async_copy(gather_src(q), rows_v.at[q], gsem[q])

    def step(s, q):
        pltpu.make_async_copy(gather_src(s), rows_v.at[q], gsem[q]).wait()
        pltpu.async_copy(rows_v.at[q], out_dst(s), osem[q])
        q2 = (q + 2) % NRING

        @pl.when(s + 2 < SEQ)
        def _():
            @pl.when(s >= 2)
            def _():
                pltpu.make_async_copy(
                    rows_v.at[q2], out_dst(s - 2), osem[q2]).wait()
            pltpu.async_copy(gather_src(s + 2), rows_v.at[q2], gsem[q2])

    def quad_body(p, _):
        for q in range(NRING):
            step(p * NRING + q, q)
        return ()

    lax.fori_loop(0, SEQ // NRING, quad_body, ())
    for q in range(SEQ % NRING):
        step(SEQ - SEQ % NRING + q, q)

    for i in range(NRING):
        s = SEQ - NRING + i
        pltpu.make_async_copy(
            rows_v.at[s % NRING], out_dst(s), osem[s % NRING]).wait()


def kernel(token_ids, weight):
    out_t = _sc_gather_t(token_ids.T, weight)
    return jnp.transpose(out_t, (1, 0, 2))
